# baseline (speedup 1.0000x reference)
import functools

import jax
import jax.numpy as jnp
from jax import lax
from jax.experimental import pallas as pl
from jax.experimental.pallas import tpu as pltpu
from jax.experimental.pallas import tpu_sc as plsc

DIM = 64
WIDE = 2 * DIM
LANES = 16
CHUNK = 128


@functools.lru_cache(maxsize=None)
def _build(B):
    info = plsc.get_sparse_core_info()
    NC, NS = info.num_cores, info.num_subcores
    NW = NC * NS
    assert B % (NW * CHUNK) == 0
    b_per_w = B // NW
    n_ch = b_per_w // CHUNK
    n_chunks = 2 * n_ch
    mesh = plsc.VectorSubcoreMesh(core_axis_name="c", subcore_axis_name="s")

    @functools.partial(
        pl.kernel,
        mesh=mesh,
        compiler_params=pltpu.CompilerParams(needs_layout_passes=False),
        out_type=(
            jax.ShapeDtypeStruct((B,), jnp.float32),
            jax.ShapeDtypeStruct((B,), jnp.float32),
            jax.ShapeDtypeStruct((B,), jnp.float32),
        ),
        scratch_types=[
            pltpu.VMEM((2, 3, n_ch, CHUNK), jnp.int32),
            pltpu.VMEM((2, 3, n_ch, CHUNK), jnp.int32),
            pltpu.VMEM((2, CHUNK, WIDE), jnp.float32),
            pltpu.VMEM((2, CHUNK, WIDE), jnp.float32),
            pltpu.VMEM((2, CHUNK, WIDE), jnp.float32),
            pltpu.VMEM((b_per_w,), jnp.float32),
            pltpu.VMEM((b_per_w,), jnp.float32),
            pltpu.VMEM((b_per_w,), jnp.float32),
            pltpu.SemaphoreType.DMA,
        ],
    )
    def score_kernel(
        human_hbm, gmf_hbm, idx_hbm,
        neg_hbm, dm_hbm, df_hbm,
        idxv, rowv, esb, epb, eob, dmv, dfv, negv, sem,
    ):
        wid = lax.axis_index("s") * NC + lax.axis_index("c")
        base = wid * b_per_w
        lane = lax.iota(jnp.int32, 16)

        pltpu.sync_copy(idx_hbm.at[wid], idxv)

        def shift_body(i, _):
            s = pl.ds(i * 16, 16)
            for g in range(2):
                for c in range(3):
                    for j in range(n_ch):
                        rowv[g, c, j, s] = jnp.right_shift(idxv[g, c, j, s], 1)
            return _

        lax.fori_loop(0, CHUNK // 16, shift_body, None)

        def fire(k):
            g, j = divmod(k, n_ch)
            slot = k % 2
            return [
                pltpu.async_copy(
                    human_hbm.at[rowv.at[g, 0, j]], esb.at[slot], sem),
                pltpu.async_copy(
                    gmf_hbm.at[rowv.at[g, 1, j]], epb.at[slot], sem),
                pltpu.async_copy(
                    gmf_hbm.at[rowv.at[g, 2, j]], eob.at[slot], sem),
            ]

        def compute(k):
            g, j = divmod(k, n_ch)
            slot = k % 2
            es, ep, eo = esb.at[slot], epb.at[slot], eob.at[slot]

            def body(gi, _):
                rows = gi * 16 + lane
                sl = pl.ds(gi * 16, 16)
                ps = jnp.bitwise_and(idxv[g, 0, j, sl], 1) * DIM
                pp = jnp.bitwise_and(idxv[g, 1, j, sl], 1) * DIM
                po = jnp.bitwise_and(idxv[g, 2, j, sl], 1) * DIM
                acc = jnp.zeros((16,), jnp.float32)
                for d in range(DIM):
                    vs = plsc.load_gather(es, [rows, ps + d])
                    vp = plsc.load_gather(ep, [rows, pp + d])
                    vo = plsc.load_gather(eo, [rows, po + d])
                    acc = acc + vs * vp * vo
                off = pl.ds(j * CHUNK + gi * 16, 16)
                if g == 0:
                    dmv[off] = acc
                else:
                    dm = dmv[off]
                    dfv[off] = acc
                    negv[off] = acc - dm
                return _

            lax.fori_loop(0, CHUNK // 16, body, None)

        pending = fire(0)
        for k in range(n_chunks):
            for c in pending:
                c.wait()
            pending = fire(k + 1) if k + 1 < n_chunks else []
            compute(k)

        out = pl.ds(base, b_per_w)
        pltpu.sync_copy(negv, neg_hbm.at[out])
        pltpu.sync_copy(dmv, dm_hbm.at[out])
        pltpu.sync_copy(dfv, df_hbm.at[out])

    return score_kernel


def kernel(human_embeds, gmf_embeds, male_triplets, female_triplets):
    B = male_triplets.shape[0]
    info = plsc.get_sparse_core_info()
    NW = info.num_cores * info.num_subcores
    n_ch = B // (NW * CHUNK)
    human2 = human_embeds.reshape(-1, WIDE)
    gmf2 = gmf_embeds.reshape(-1, WIDE)
    tri = jnp.stack([male_triplets, female_triplets])
    idx = tri.reshape(2, NW, n_ch, CHUNK, 3).transpose(1, 0, 4, 2, 3)
    neg, d_male, d_female = _build(B)(human2, gmf2, idx)
    return (neg, d_male, d_female)
